# baseline (speedup 1.0000x reference)
import functools

import jax
import jax.numpy as jnp
from jax import lax
from jax.experimental import pallas as pl
from jax.experimental.pallas import tpu as pltpu
from jax.experimental.pallas import tpu_sc as plsc

_S = 512
_K = 32
_QC = 64
_SCCH = 128


def _fps_body(xt_ref, out_ref):
    B, _, N = xt_ref.shape
    S = out_ref.shape[2]
    x = xt_ref[:, 0, :]
    y = xt_ref[:, 1, :]
    z = xt_ref[:, 2, :]
    iota = jax.lax.broadcasted_iota(jnp.int32, (B, N), 1)
    sio = jax.lax.broadcasted_iota(jnp.int32, (B, S), 1)

    def body(i, carry):
        dists, nxt, acc = carry
        eq = iota == nxt
        px = jnp.sum(jnp.where(eq, x, 0.0), axis=1, keepdims=True)
        py = jnp.sum(jnp.where(eq, y, 0.0), axis=1, keepdims=True)
        pz = jnp.sum(jnp.where(eq, z, 0.0), axis=1, keepdims=True)
        dx = x - px
        dy = y - py
        dz = z - pz
        d = dx * dx + dy * dy + dz * dz
        dists = jnp.minimum(dists, d)
        nn = jnp.argmax(dists, axis=1, keepdims=True).astype(jnp.int32)
        acc = jnp.where(sio == i, nn, acc)
        return dists, nn, acc

    init = (jnp.full((B, N), jnp.inf, jnp.float32),
            jnp.zeros((B, 1), jnp.int32),
            jnp.zeros((B, S), jnp.int32))
    _, _, acc = jax.lax.fori_loop(1, S, body, init)
    out_ref[:, 0, :] = acc


def _knn_body(comb_ref, xt_ref, fidx_ref, nrows_ref, kidx_ref):
    _, N, C = comb_ref.shape
    S = nrows_ref.shape[1]
    K = kidx_ref.shape[2]
    comb = comb_ref[0]
    fidx = fidx_ref[0]
    niota = jax.lax.broadcasted_iota(jnp.int32, (N, S), 0)
    ohT = jnp.where(niota == fidx, 1.0, 0.0).astype(jnp.float32)
    nrows = jax.lax.dot_general(ohT, comb, (((0,), (0,)), ((), ())),
                                precision=jax.lax.Precision.HIGHEST,
                                preferred_element_type=jnp.float32)
    nrows_ref[0] = nrows
    x = xt_ref[0, 0:1, :]
    y = xt_ref[0, 1:2, :]
    z = xt_ref[0, 2:3, :]
    qx = nrows[:, C - 3:C - 2]
    qy = nrows[:, C - 2:C - 1]
    qz = nrows[:, C - 1:C]
    dxm = qx - x
    dym = qy - y
    dzm = qz - z
    d = dxm * dxm + dym * dym + dzm * dzm
    iota_n = jax.lax.broadcasted_iota(jnp.int32, (S, N), 1)
    kio = jax.lax.broadcasted_iota(jnp.int32, (S, K), 1)

    def body(j, carry):
        dcur, acc = carry
        e = jnp.argmin(dcur, axis=1, keepdims=True).astype(jnp.int32)
        acc = jnp.where(kio == j, e, acc)
        dcur = jnp.where(iota_n == e, jnp.inf, dcur)
        return dcur, acc

    _, acc = jax.lax.fori_loop(0, K, body, (d, jnp.zeros((S, K), jnp.int32)))
    kidx_ref[0] = acc + pl.program_id(0) * N


def _sc_gather_body(pts_ref, xh_ref, yh_ref, zh_ref, gidx_ref,
                    gp_ref, gx_ref, gy_ref, gz_ref,
                    xt_v, yt_v, zt_v, idx_v, rows_v, xb_v, yb_v, zb_v, sem):
    R = gidx_ref.shape[0]
    ch = idx_v.shape[0]
    nc = 2
    nw = 32
    per_w = R // nw
    chunks = per_w // ch
    wid = lax.axis_index("s") * nc + lax.axis_index("c")

    pltpu.sync_copy(xh_ref, xt_v)
    pltpu.sync_copy(yh_ref, yt_v)
    pltpu.sync_copy(zh_ref, zt_v)

    def body(i, carry):
        base = wid * per_w + i * ch
        pltpu.sync_copy(gidx_ref.at[pl.ds(base, ch)], idx_v)
        cp = pltpu.async_copy(pts_ref.at[idx_v], rows_v, sem)
        for t in range(ch // 16):
            iv = idx_v[pl.ds(16 * t, 16)]
            xb_v[pl.ds(16 * t, 16)] = plsc.load_gather(xt_v, [iv])
            yb_v[pl.ds(16 * t, 16)] = plsc.load_gather(yt_v, [iv])
            zb_v[pl.ds(16 * t, 16)] = plsc.load_gather(zt_v, [iv])
        pltpu.sync_copy(xb_v, gx_ref.at[pl.ds(base, ch)])
        pltpu.sync_copy(yb_v, gy_ref.at[pl.ds(base, ch)])
        pltpu.sync_copy(zb_v, gz_ref.at[pl.ds(base, ch)])
        cp.wait()
        pltpu.sync_copy(rows_v, gp_ref.at[pl.ds(base, ch)])
        return carry

    jax.lax.fori_loop(0, chunks, body, 0)


def _stats_body(gp_ref, gx_ref, gy_ref, gz_ref, mean_ref, vpart_ref):
    _, QC, K, D = gp_ref.shape
    NCH = vpart_ref.shape[2]
    c = pl.program_id(1)
    gp = gp_ref[0]
    mup = jnp.mean(gp, axis=1)
    vp = gp - mup[:, None, :]
    s = jnp.sum(vp * vp)
    mus = []
    for r in (gx_ref, gy_ref, gz_ref):
        gc = r[0]
        muc = jnp.mean(gc, axis=1, keepdims=True)
        vc = gc - muc
        s = s + jnp.sum(vc * vc)
        mus.append(muc)
    mean_ref[0] = jnp.concatenate([mup] + mus, axis=1)
    lio = jax.lax.broadcasted_iota(jnp.int32, (1, NCH), 1)

    @pl.when(c == 0)
    def _():
        vpart_ref[0] = jnp.zeros((1, NCH), jnp.float32)

    vpart_ref[0] = vpart_ref[0] + jnp.where(lio == c, s, 0.0)


def _out_body(gp_ref, gx_ref, gy_ref, gz_ref, mean_ref, vparts_ref,
              nrows_ref, ab_ref, out_ref):
    _, QC, K, D = gp_ref.shape
    C = mean_ref.shape[2]
    NCH = vparts_ref.shape[2]
    M = NCH * QC * K * C
    s2 = jnp.sum(vparts_ref[0, 0, :])
    std = jnp.sqrt(s2 / (M - 1))
    scale = 1.0 / (std + 1e-5)
    mu = mean_ref[0]
    alpha = ab_ref[0:1, :]
    beta = ab_ref[1:2, :]
    normp = (gp_ref[0] - mu[:, None, 0:D]) * scale
    normp = normp * alpha[None, :, 0:D] + beta[None, :, 0:D]
    parts = [normp]
    for j, r in enumerate((gx_ref, gy_ref, gz_ref)):
        gc = r[0]
        nc = (gc - mu[:, D + j:D + j + 1]) * scale
        nc = nc * ab_ref[0:1, D + j:D + j + 1] + ab_ref[1:2, D + j:D + j + 1]
        parts.append(nc[:, :, None])
    rep = nrows_ref[0][:, 0:D]
    parts.append(jnp.broadcast_to(rep[:, None, :], (QC, K, D)))
    out_ref[0] = jnp.concatenate(parts, axis=-1)


def _half_pipeline(xyz, points, comb, xt, fidx, ab, S, K, QC):
    B, N, _ = xyz.shape
    D = points.shape[2]
    C = D + 3
    NCH = S // QC

    nrows, kidx = pl.pallas_call(
        _knn_body,
        grid=(B,),
        in_specs=[pl.BlockSpec((1, N, C), lambda b: (b, 0, 0)),
                  pl.BlockSpec((1, 3, N), lambda b: (b, 0, 0)),
                  pl.BlockSpec((1, 1, S), lambda b: (b, 0, 0))],
        out_specs=[pl.BlockSpec((1, S, C), lambda b: (b, 0, 0)),
                   pl.BlockSpec((1, S, K), lambda b: (b, 0, 0))],
        out_shape=[jax.ShapeDtypeStruct((B, S, C), jnp.float32),
                   jax.ShapeDtypeStruct((B, S, K), jnp.int32)],
    )(comb, xt, fidx)

    R = B * S * K
    pts_flat = points.reshape(B * N, D)
    xh = xyz[:, :, 0].reshape(B * N)
    yh = xyz[:, :, 1].reshape(B * N)
    zh = xyz[:, :, 2].reshape(B * N)
    gidx = kidx.reshape(R)
    mesh = plsc.VectorSubcoreMesh(core_axis_name="c", subcore_axis_name="s")
    gp_flat, gx, gy, gz = pl.kernel(
        _sc_gather_body,
        mesh=mesh,
        compiler_params=pltpu.CompilerParams(needs_layout_passes=False),
        out_type=[jax.ShapeDtypeStruct((R, D), jnp.float32),
                  jax.ShapeDtypeStruct((R,), jnp.float32),
                  jax.ShapeDtypeStruct((R,), jnp.float32),
                  jax.ShapeDtypeStruct((R,), jnp.float32)],
        scratch_types=[pltpu.VMEM((B * N,), jnp.float32),
                       pltpu.VMEM((B * N,), jnp.float32),
                       pltpu.VMEM((B * N,), jnp.float32),
                       pltpu.VMEM((_SCCH,), jnp.int32),
                       pltpu.VMEM((_SCCH, D), jnp.float32),
                       pltpu.VMEM((_SCCH,), jnp.float32),
                       pltpu.VMEM((_SCCH,), jnp.float32),
                       pltpu.VMEM((_SCCH,), jnp.float32),
                       pltpu.SemaphoreType.DMA],
    )(pts_flat, xh, yh, zh, gidx)
    gp = gp_flat.reshape(B, S, K, D)
    gx3 = gx.reshape(B, S, K)
    gy3 = gy.reshape(B, S, K)
    gz3 = gz.reshape(B, S, K)

    xspec = pl.BlockSpec((1, QC, K), lambda b, c: (b, c, 0))
    mean, vparts = pl.pallas_call(
        _stats_body,
        grid=(B, NCH),
        in_specs=[pl.BlockSpec((1, QC, K, D), lambda b, c: (b, c, 0, 0)),
                  xspec, xspec, xspec],
        out_specs=[pl.BlockSpec((1, QC, C), lambda b, c: (b, c, 0)),
                   pl.BlockSpec((1, 1, NCH), lambda b, c: (b, 0, 0))],
        out_shape=[jax.ShapeDtypeStruct((B, S, C), jnp.float32),
                   jax.ShapeDtypeStruct((B, 1, NCH), jnp.float32)],
    )(gp, gx3, gy3, gz3)

    out = pl.pallas_call(
        _out_body,
        grid=(B, NCH),
        in_specs=[pl.BlockSpec((1, QC, K, D), lambda b, c: (b, c, 0, 0)),
                  xspec, xspec, xspec,
                  pl.BlockSpec((1, QC, C), lambda b, c: (b, c, 0)),
                  pl.BlockSpec((1, 1, NCH), lambda b, c: (b, 0, 0)),
                  pl.BlockSpec((1, QC, C), lambda b, c: (b, c, 0)),
                  pl.BlockSpec((2, C), lambda b, c: (0, 0))],
        out_specs=pl.BlockSpec((1, QC, K, C + D), lambda b, c: (b, c, 0, 0)),
        out_shape=jax.ShapeDtypeStruct((B, S, K, C + D), jnp.float32),
    )(gp, gx3, gy3, gz3, mean, vparts, nrows, ab)

    new_xyz = nrows[:, :, D:C]
    return new_xyz, out


def kernel(xyz, points, affine_alpha, affine_beta):
    B, N, _ = xyz.shape
    D = points.shape[2]
    S, K, QC = _S, _K, _QC
    C = D + 3

    xt = jnp.transpose(xyz, (0, 2, 1))
    comb = jnp.concatenate([points, xyz], axis=2)
    ab = jnp.concatenate([affine_alpha.reshape(1, C),
                          affine_beta.reshape(1, C)], axis=0)

    fidx = pl.pallas_call(
        _fps_body,
        out_shape=jax.ShapeDtypeStruct((B, 1, S), jnp.int32),
    )(xt)

    hb = B // 2
    outs = []
    for lo, hi in ((0, hb), (hb, B)):
        outs.append(_half_pipeline(
            xyz[lo:hi], points[lo:hi], comb[lo:hi], xt[lo:hi],
            fidx[lo:hi], ab, S, K, QC))

    new_xyz = jnp.concatenate([o[0] for o in outs], axis=0)
    out = jnp.concatenate([o[1] for o in outs], axis=0)
    return new_xyz, out


# speedup vs baseline: 1.0732x; 1.0732x over previous
import functools

import jax
import jax.numpy as jnp
from jax import lax
from jax.experimental import pallas as pl
from jax.experimental.pallas import tpu as pltpu
from jax.experimental.pallas import tpu_sc as plsc

_S = 512
_K = 32
_QC = 64
_SCCH = 128


def _fps_body(xt_ref, out_ref):
    B, _, N = xt_ref.shape
    S = out_ref.shape[2]
    x = xt_ref[:, 0, :]
    y = xt_ref[:, 1, :]
    z = xt_ref[:, 2, :]
    iota = jax.lax.broadcasted_iota(jnp.int32, (B, N), 1)
    sio = jax.lax.broadcasted_iota(jnp.int32, (B, S), 1)

    def body(i, carry):
        dists, nxt, acc = carry
        eq = iota == nxt
        px = jnp.sum(jnp.where(eq, x, 0.0), axis=1, keepdims=True)
        py = jnp.sum(jnp.where(eq, y, 0.0), axis=1, keepdims=True)
        pz = jnp.sum(jnp.where(eq, z, 0.0), axis=1, keepdims=True)
        dx = x - px
        dy = y - py
        dz = z - pz
        d = dx * dx + dy * dy + dz * dz
        dists = jnp.minimum(dists, d)
        nn = jnp.argmax(dists, axis=1, keepdims=True).astype(jnp.int32)
        acc = jnp.where(sio == i, nn, acc)
        return dists, nn, acc

    init = (jnp.full((B, N), jnp.inf, jnp.float32),
            jnp.zeros((B, 1), jnp.int32),
            jnp.zeros((B, S), jnp.int32))
    _, _, acc = jax.lax.fori_loop(1, S, body, init)
    out_ref[:, 0, :] = acc


def _knn_body(comb_ref, xt_ref, fidx_ref, nrows_ref, kidx_ref):
    _, N, C = comb_ref.shape
    S = nrows_ref.shape[1]
    K = kidx_ref.shape[2]
    comb = comb_ref[0]
    fidx = fidx_ref[0]
    niota = jax.lax.broadcasted_iota(jnp.int32, (N, S), 0)
    ohT = jnp.where(niota == fidx, 1.0, 0.0).astype(jnp.float32)
    nrows = jax.lax.dot_general(ohT, comb, (((0,), (0,)), ((), ())),
                                precision=jax.lax.Precision.HIGHEST,
                                preferred_element_type=jnp.float32)
    nrows_ref[0] = nrows
    x = xt_ref[0, 0:1, :]
    y = xt_ref[0, 1:2, :]
    z = xt_ref[0, 2:3, :]
    qx = nrows[:, C - 3:C - 2]
    qy = nrows[:, C - 2:C - 1]
    qz = nrows[:, C - 1:C]
    dxm = qx - x
    dym = qy - y
    dzm = qz - z
    d = dxm * dxm + dym * dym + dzm * dzm
    iota_n = jax.lax.broadcasted_iota(jnp.int32, (S, N), 1)
    kio = jax.lax.broadcasted_iota(jnp.int32, (S, K), 1)

    def body(j, carry):
        dcur, acc = carry
        e = jnp.argmin(dcur, axis=1, keepdims=True).astype(jnp.int32)
        acc = jnp.where(kio == j, e, acc)
        dcur = jnp.where(iota_n == e, jnp.inf, dcur)
        return dcur, acc

    _, acc = jax.lax.fori_loop(0, K, body, (d, jnp.zeros((S, K), jnp.int32)))
    kidx_ref[0] = acc + pl.program_id(0) * N


def _sc_gather_body(pts_ref, xh_ref, yh_ref, zh_ref, gidx_ref,
                    gp_ref, gx_ref, gy_ref, gz_ref,
                    xt_v, yt_v, zt_v, idx_v, rows_v, xb_v, yb_v, zb_v, sem):
    R = gidx_ref.shape[0]
    ch = idx_v.shape[0]
    nc = 2
    nw = 32
    per_w = R // nw
    chunks = per_w // ch
    wid = lax.axis_index("s") * nc + lax.axis_index("c")

    pltpu.sync_copy(xh_ref, xt_v)
    pltpu.sync_copy(yh_ref, yt_v)
    pltpu.sync_copy(zh_ref, zt_v)

    def body(i, carry):
        base = wid * per_w + i * ch
        pltpu.sync_copy(gidx_ref.at[pl.ds(base, ch)], idx_v)
        cp = pltpu.async_copy(pts_ref.at[idx_v], rows_v, sem)
        for t in range(ch // 16):
            iv = idx_v[pl.ds(16 * t, 16)]
            xb_v[pl.ds(16 * t, 16)] = plsc.load_gather(xt_v, [iv])
            yb_v[pl.ds(16 * t, 16)] = plsc.load_gather(yt_v, [iv])
            zb_v[pl.ds(16 * t, 16)] = plsc.load_gather(zt_v, [iv])
        pltpu.sync_copy(xb_v, gx_ref.at[pl.ds(base, ch)])
        pltpu.sync_copy(yb_v, gy_ref.at[pl.ds(base, ch)])
        pltpu.sync_copy(zb_v, gz_ref.at[pl.ds(base, ch)])
        cp.wait()
        pltpu.sync_copy(rows_v, gp_ref.at[pl.ds(base, ch)])
        return carry

    jax.lax.fori_loop(0, chunks, body, 0)


def _stats_body(gp_ref, gx_ref, gy_ref, gz_ref, mean_ref, vpart_ref):
    _, QC, K, D = gp_ref.shape
    NCH = vpart_ref.shape[2]
    c = pl.program_id(1)
    gp = gp_ref[0]
    mup = jnp.mean(gp, axis=1)
    vp = gp - mup[:, None, :]
    s = jnp.sum(vp * vp)
    mus = []
    for r in (gx_ref, gy_ref, gz_ref):
        gc = r[0]
        muc = jnp.mean(gc, axis=1, keepdims=True)
        vc = gc - muc
        s = s + jnp.sum(vc * vc)
        mus.append(muc)
    mean_ref[0] = jnp.concatenate([mup] + mus, axis=1)
    lio = jax.lax.broadcasted_iota(jnp.int32, (1, NCH), 1)

    @pl.when(c == 0)
    def _():
        vpart_ref[0] = jnp.zeros((1, NCH), jnp.float32)

    vpart_ref[0] = vpart_ref[0] + jnp.where(lio == c, s, 0.0)


def _out_body(gp_ref, gx_ref, gy_ref, gz_ref, mean_ref, vparts_ref,
              nrows_ref, ab_ref, out_ref):
    _, QC, K, D = gp_ref.shape
    C = mean_ref.shape[2]
    NCH = vparts_ref.shape[2]
    M = NCH * QC * K * C
    s2 = jnp.sum(vparts_ref[0, 0, :])
    std = jnp.sqrt(s2 / (M - 1))
    scale = 1.0 / (std + 1e-5)
    mu = mean_ref[0]
    alpha = ab_ref[0:1, :]
    beta = ab_ref[1:2, :]
    normp = (gp_ref[0] - mu[:, None, 0:D]) * scale
    normp = normp * alpha[None, :, 0:D] + beta[None, :, 0:D]
    parts = [normp]
    for j, r in enumerate((gx_ref, gy_ref, gz_ref)):
        gc = r[0]
        nc = (gc - mu[:, D + j:D + j + 1]) * scale
        nc = nc * ab_ref[0:1, D + j:D + j + 1] + ab_ref[1:2, D + j:D + j + 1]
        parts.append(nc[:, :, None])
    rep = nrows_ref[0][:, 0:D]
    parts.append(jnp.broadcast_to(rep[:, None, :], (QC, K, D)))
    out_ref[0] = jnp.concatenate(parts, axis=-1)


def _fused_out_body(gp_ref, gx_ref, gy_ref, gz_ref, nrows_ref, ab_ref,
                    out_ref, mean_scr, scale_scr):
    _, S, K, D = gp_ref.shape
    C = nrows_ref.shape[2]
    QC = out_ref.shape[1]
    M = S * K * C
    c = pl.program_id(1)

    @pl.when(c == 0)
    def _():
        gp = gp_ref[0]
        mup = jnp.mean(gp, axis=1)
        vp = gp - mup[:, None, :]
        s2 = jnp.sum(vp * vp)
        mucs = []
        for r in (gx_ref, gy_ref, gz_ref):
            gc = r[0]
            muc = jnp.mean(gc, axis=1, keepdims=True)
            vc = gc - muc
            s2 = s2 + jnp.sum(vc * vc)
            mucs.append(muc)
        std = jnp.sqrt(s2 / (M - 1))
        mean_scr[:, :] = jnp.concatenate([mup] + mucs, axis=1)
        scale_scr[0, 0] = 1.0 / (std + 1e-5)

    scale = scale_scr[0, 0]
    ab = ab_ref[:, :]
    mu = mean_scr[pl.ds(QC * c, QC), :]
    gpc = gp_ref[0, pl.ds(QC * c, QC)]
    normp = (gpc - mu[:, None, 0:D]) * scale
    normp = normp * ab[0:1, None, 0:D] + ab[1:2, None, 0:D]
    parts = [normp]
    for j, r in enumerate((gx_ref, gy_ref, gz_ref)):
        gc = r[0, pl.ds(QC * c, QC)]
        nc = (gc - mu[:, D + j:D + j + 1]) * scale
        nc = nc * ab[0:1, D + j:D + j + 1] + ab[1:2, D + j:D + j + 1]
        parts.append(nc[:, :, None])
    rep = nrows_ref[0, pl.ds(QC * c, QC), 0:D]
    parts.append(jnp.broadcast_to(rep[:, None, :], (QC, K, D)))
    out_ref[0] = jnp.concatenate(parts, axis=-1)


def _half_pipeline(xyz, points, comb, xt, fidx, ab, S, K, QC):
    B, N, _ = xyz.shape
    D = points.shape[2]
    C = D + 3
    NCH = S // QC

    nrows, kidx = pl.pallas_call(
        _knn_body,
        grid=(B,),
        in_specs=[pl.BlockSpec((1, N, C), lambda b: (b, 0, 0)),
                  pl.BlockSpec((1, 3, N), lambda b: (b, 0, 0)),
                  pl.BlockSpec((1, 1, S), lambda b: (b, 0, 0))],
        out_specs=[pl.BlockSpec((1, S, C), lambda b: (b, 0, 0)),
                   pl.BlockSpec((1, S, K), lambda b: (b, 0, 0))],
        out_shape=[jax.ShapeDtypeStruct((B, S, C), jnp.float32),
                   jax.ShapeDtypeStruct((B, S, K), jnp.int32)],
    )(comb, xt, fidx)

    R = B * S * K
    pts_flat = points.reshape(B * N, D)
    xh = xyz[:, :, 0].reshape(B * N)
    yh = xyz[:, :, 1].reshape(B * N)
    zh = xyz[:, :, 2].reshape(B * N)
    gidx = kidx.reshape(R)
    mesh = plsc.VectorSubcoreMesh(core_axis_name="c", subcore_axis_name="s")
    gp_flat, gx, gy, gz = pl.kernel(
        _sc_gather_body,
        mesh=mesh,
        compiler_params=pltpu.CompilerParams(needs_layout_passes=False),
        out_type=[jax.ShapeDtypeStruct((R, D), jnp.float32),
                  jax.ShapeDtypeStruct((R,), jnp.float32),
                  jax.ShapeDtypeStruct((R,), jnp.float32),
                  jax.ShapeDtypeStruct((R,), jnp.float32)],
        scratch_types=[pltpu.VMEM((B * N,), jnp.float32),
                       pltpu.VMEM((B * N,), jnp.float32),
                       pltpu.VMEM((B * N,), jnp.float32),
                       pltpu.VMEM((_SCCH,), jnp.int32),
                       pltpu.VMEM((_SCCH, D), jnp.float32),
                       pltpu.VMEM((_SCCH,), jnp.float32),
                       pltpu.VMEM((_SCCH,), jnp.float32),
                       pltpu.VMEM((_SCCH,), jnp.float32),
                       pltpu.SemaphoreType.DMA],
    )(pts_flat, xh, yh, zh, gidx)
    gp = gp_flat.reshape(B, S, K, D)
    gx3 = gx.reshape(B, S, K)
    gy3 = gy.reshape(B, S, K)
    gz3 = gz.reshape(B, S, K)

    xspec = pl.BlockSpec((1, S, K), lambda b, c: (b, 0, 0))
    out = pl.pallas_call(
        _fused_out_body,
        grid=(B, NCH),
        in_specs=[pl.BlockSpec((1, S, K, D), lambda b, c: (b, 0, 0, 0)),
                  xspec, xspec, xspec,
                  pl.BlockSpec((1, S, C), lambda b, c: (b, 0, 0)),
                  pl.BlockSpec((2, C), lambda b, c: (0, 0))],
        out_specs=pl.BlockSpec((1, QC, K, C + D), lambda b, c: (b, c, 0, 0)),
        out_shape=jax.ShapeDtypeStruct((B, S, K, C + D), jnp.float32),
        scratch_shapes=[pltpu.VMEM((S, C), jnp.float32),
                        pltpu.SMEM((1, 1), jnp.float32)],
    )(gp, gx3, gy3, gz3, nrows, ab)

    new_xyz = nrows[:, :, D:C]
    return new_xyz, out


def kernel(xyz, points, affine_alpha, affine_beta):
    B, N, _ = xyz.shape
    D = points.shape[2]
    S, K, QC = _S, _K, _QC
    C = D + 3

    xt = jnp.transpose(xyz, (0, 2, 1))
    comb = jnp.concatenate([points, xyz], axis=2)
    ab = jnp.concatenate([affine_alpha.reshape(1, C),
                          affine_beta.reshape(1, C)], axis=0)

    fidx = pl.pallas_call(
        _fps_body,
        out_shape=jax.ShapeDtypeStruct((B, 1, S), jnp.int32),
    )(xt)

    return _half_pipeline(xyz, points, comb, xt, fidx, ab, S, K, QC)


# speedup vs baseline: 1.0746x; 1.0013x over previous
import functools

import jax
import jax.numpy as jnp
from jax import lax
from jax.experimental import pallas as pl
from jax.experimental.pallas import tpu as pltpu
from jax.experimental.pallas import tpu_sc as plsc

_S = 512
_K = 32
_QC = 128
_SCCH = 128


def _fps_body(xt_ref, out_ref):
    B, _, N = xt_ref.shape
    S = out_ref.shape[2]
    x = xt_ref[:, 0, :]
    y = xt_ref[:, 1, :]
    z = xt_ref[:, 2, :]
    iota = jax.lax.broadcasted_iota(jnp.int32, (B, N), 1)
    sio = jax.lax.broadcasted_iota(jnp.int32, (B, S), 1)

    def body(i, carry):
        dists, nxt, acc = carry
        eq = iota == nxt
        px = jnp.sum(jnp.where(eq, x, 0.0), axis=1, keepdims=True)
        py = jnp.sum(jnp.where(eq, y, 0.0), axis=1, keepdims=True)
        pz = jnp.sum(jnp.where(eq, z, 0.0), axis=1, keepdims=True)
        dx = x - px
        dy = y - py
        dz = z - pz
        d = dx * dx + dy * dy + dz * dz
        dists = jnp.minimum(dists, d)
        nn = jnp.argmax(dists, axis=1, keepdims=True).astype(jnp.int32)
        acc = jnp.where(sio == i, nn, acc)
        return dists, nn, acc

    init = (jnp.full((B, N), jnp.inf, jnp.float32),
            jnp.zeros((B, 1), jnp.int32),
            jnp.zeros((B, S), jnp.int32))
    _, _, acc = jax.lax.fori_loop(1, S, body, init)
    out_ref[:, 0, :] = acc


def _knn_body(comb_ref, xt_ref, fidx_ref, nrows_ref, kidx_ref):
    _, N, C = comb_ref.shape
    S = nrows_ref.shape[1]
    K = kidx_ref.shape[2]
    comb = comb_ref[0]
    fidx = fidx_ref[0]
    niota = jax.lax.broadcasted_iota(jnp.int32, (N, S), 0)
    ohT = jnp.where(niota == fidx, 1.0, 0.0).astype(jnp.float32)
    nrows = jax.lax.dot_general(ohT, comb, (((0,), (0,)), ((), ())),
                                precision=jax.lax.Precision.HIGHEST,
                                preferred_element_type=jnp.float32)
    nrows_ref[0] = nrows
    x = xt_ref[0, 0:1, :]
    y = xt_ref[0, 1:2, :]
    z = xt_ref[0, 2:3, :]
    qx = nrows[:, C - 3:C - 2]
    qy = nrows[:, C - 2:C - 1]
    qz = nrows[:, C - 1:C]
    dxm = qx - x
    dym = qy - y
    dzm = qz - z
    d = dxm * dxm + dym * dym + dzm * dzm
    iota_n = jax.lax.broadcasted_iota(jnp.int32, (S, N), 1)
    kio = jax.lax.broadcasted_iota(jnp.int32, (S, K), 1)

    def body(j, carry):
        dcur, acc = carry
        e = jnp.argmin(dcur, axis=1, keepdims=True).astype(jnp.int32)
        acc = jnp.where(kio == j, e, acc)
        dcur = jnp.where(iota_n == e, jnp.inf, dcur)
        return dcur, acc

    _, acc = jax.lax.fori_loop(0, K, body, (d, jnp.zeros((S, K), jnp.int32)))
    kidx_ref[0] = acc + pl.program_id(0) * N


def _sc_gather_body(pts_ref, xh_ref, yh_ref, zh_ref, gidx_ref,
                    gp_ref, gx_ref, gy_ref, gz_ref,
                    xt_v, yt_v, zt_v, idx_v, rows_v, xb_v, yb_v, zb_v, sem):
    R = gidx_ref.shape[0]
    ch = idx_v.shape[0]
    nc = 2
    nw = 32
    per_w = R // nw
    chunks = per_w // ch
    wid = lax.axis_index("s") * nc + lax.axis_index("c")

    pltpu.sync_copy(xh_ref, xt_v)
    pltpu.sync_copy(yh_ref, yt_v)
    pltpu.sync_copy(zh_ref, zt_v)

    def body(i, carry):
        base = wid * per_w + i * ch
        pltpu.sync_copy(gidx_ref.at[pl.ds(base, ch)], idx_v)
        cp = pltpu.async_copy(pts_ref.at[idx_v], rows_v, sem)
        for t in range(ch // 16):
            iv = idx_v[pl.ds(16 * t, 16)]
            xb_v[pl.ds(16 * t, 16)] = plsc.load_gather(xt_v, [iv])
            yb_v[pl.ds(16 * t, 16)] = plsc.load_gather(yt_v, [iv])
            zb_v[pl.ds(16 * t, 16)] = plsc.load_gather(zt_v, [iv])
        pltpu.sync_copy(xb_v, gx_ref.at[pl.ds(base, ch)])
        pltpu.sync_copy(yb_v, gy_ref.at[pl.ds(base, ch)])
        pltpu.sync_copy(zb_v, gz_ref.at[pl.ds(base, ch)])
        cp.wait()
        pltpu.sync_copy(rows_v, gp_ref.at[pl.ds(base, ch)])
        return carry

    jax.lax.fori_loop(0, chunks, body, 0)


def _stats_body(gp_ref, gx_ref, gy_ref, gz_ref, mean_ref, vpart_ref):
    _, QC, K, D = gp_ref.shape
    NCH = vpart_ref.shape[2]
    c = pl.program_id(1)
    gp = gp_ref[0]
    mup = jnp.mean(gp, axis=1)
    vp = gp - mup[:, None, :]
    s = jnp.sum(vp * vp)
    mus = []
    for r in (gx_ref, gy_ref, gz_ref):
        gc = r[0]
        muc = jnp.mean(gc, axis=1, keepdims=True)
        vc = gc - muc
        s = s + jnp.sum(vc * vc)
        mus.append(muc)
    mean_ref[0] = jnp.concatenate([mup] + mus, axis=1)
    lio = jax.lax.broadcasted_iota(jnp.int32, (1, NCH), 1)

    @pl.when(c == 0)
    def _():
        vpart_ref[0] = jnp.zeros((1, NCH), jnp.float32)

    vpart_ref[0] = vpart_ref[0] + jnp.where(lio == c, s, 0.0)


def _out_body(gp_ref, gx_ref, gy_ref, gz_ref, mean_ref, vparts_ref,
              nrows_ref, ab_ref, out_ref):
    _, QC, K, D = gp_ref.shape
    C = mean_ref.shape[2]
    NCH = vparts_ref.shape[2]
    M = NCH * QC * K * C
    s2 = jnp.sum(vparts_ref[0, 0, :])
    std = jnp.sqrt(s2 / (M - 1))
    scale = 1.0 / (std + 1e-5)
    mu = mean_ref[0]
    alpha = ab_ref[0:1, :]
    beta = ab_ref[1:2, :]
    normp = (gp_ref[0] - mu[:, None, 0:D]) * scale
    normp = normp * alpha[None, :, 0:D] + beta[None, :, 0:D]
    parts = [normp]
    for j, r in enumerate((gx_ref, gy_ref, gz_ref)):
        gc = r[0]
        nc = (gc - mu[:, D + j:D + j + 1]) * scale
        nc = nc * ab_ref[0:1, D + j:D + j + 1] + ab_ref[1:2, D + j:D + j + 1]
        parts.append(nc[:, :, None])
    rep = nrows_ref[0][:, 0:D]
    parts.append(jnp.broadcast_to(rep[:, None, :], (QC, K, D)))
    out_ref[0] = jnp.concatenate(parts, axis=-1)


def _fused_out_body(gp_ref, gx_ref, gy_ref, gz_ref, nrows_ref, ab_ref,
                    out_ref, mean_scr, scale_scr):
    _, S, K, D = gp_ref.shape
    C = nrows_ref.shape[2]
    QC = out_ref.shape[1]
    M = S * K * C
    c = pl.program_id(1)

    @pl.when(c == 0)
    def _():
        gp = gp_ref[0]
        mup = jnp.mean(gp, axis=1)
        vp = gp - mup[:, None, :]
        s2 = jnp.sum(vp * vp)
        mucs = []
        for r in (gx_ref, gy_ref, gz_ref):
            gc = r[0]
            muc = jnp.mean(gc, axis=1, keepdims=True)
            vc = gc - muc
            s2 = s2 + jnp.sum(vc * vc)
            mucs.append(muc)
        std = jnp.sqrt(s2 / (M - 1))
        mean_scr[:, :] = jnp.concatenate([mup] + mucs, axis=1)
        scale_scr[0, 0] = 1.0 / (std + 1e-5)

    scale = scale_scr[0, 0]
    ab = ab_ref[:, :]
    mu = mean_scr[pl.ds(QC * c, QC), :]
    gpc = gp_ref[0, pl.ds(QC * c, QC)]
    normp = (gpc - mu[:, None, 0:D]) * scale
    normp = normp * ab[0:1, None, 0:D] + ab[1:2, None, 0:D]
    parts = [normp]
    for j, r in enumerate((gx_ref, gy_ref, gz_ref)):
        gc = r[0, pl.ds(QC * c, QC)]
        nc = (gc - mu[:, D + j:D + j + 1]) * scale
        nc = nc * ab[0:1, D + j:D + j + 1] + ab[1:2, D + j:D + j + 1]
        parts.append(nc[:, :, None])
    rep = nrows_ref[0, pl.ds(QC * c, QC), 0:D]
    parts.append(jnp.broadcast_to(rep[:, None, :], (QC, K, D)))
    out_ref[0] = jnp.concatenate(parts, axis=-1)


def _half_pipeline(xyz, points, comb, xt, fidx, ab, S, K, QC):
    B, N, _ = xyz.shape
    D = points.shape[2]
    C = D + 3
    NCH = S // QC

    nrows, kidx = pl.pallas_call(
        _knn_body,
        grid=(B,),
        in_specs=[pl.BlockSpec((1, N, C), lambda b: (b, 0, 0)),
                  pl.BlockSpec((1, 3, N), lambda b: (b, 0, 0)),
                  pl.BlockSpec((1, 1, S), lambda b: (b, 0, 0))],
        out_specs=[pl.BlockSpec((1, S, C), lambda b: (b, 0, 0)),
                   pl.BlockSpec((1, S, K), lambda b: (b, 0, 0))],
        out_shape=[jax.ShapeDtypeStruct((B, S, C), jnp.float32),
                   jax.ShapeDtypeStruct((B, S, K), jnp.int32)],
    )(comb, xt, fidx)

    R = B * S * K
    pts_flat = points.reshape(B * N, D)
    xh = xyz[:, :, 0].reshape(B * N)
    yh = xyz[:, :, 1].reshape(B * N)
    zh = xyz[:, :, 2].reshape(B * N)
    gidx = kidx.reshape(R)
    mesh = plsc.VectorSubcoreMesh(core_axis_name="c", subcore_axis_name="s")
    gp_flat, gx, gy, gz = pl.kernel(
        _sc_gather_body,
        mesh=mesh,
        compiler_params=pltpu.CompilerParams(needs_layout_passes=False),
        out_type=[jax.ShapeDtypeStruct((R, D), jnp.float32),
                  jax.ShapeDtypeStruct((R,), jnp.float32),
                  jax.ShapeDtypeStruct((R,), jnp.float32),
                  jax.ShapeDtypeStruct((R,), jnp.float32)],
        scratch_types=[pltpu.VMEM((B * N,), jnp.float32),
                       pltpu.VMEM((B * N,), jnp.float32),
                       pltpu.VMEM((B * N,), jnp.float32),
                       pltpu.VMEM((_SCCH,), jnp.int32),
                       pltpu.VMEM((_SCCH, D), jnp.float32),
                       pltpu.VMEM((_SCCH,), jnp.float32),
                       pltpu.VMEM((_SCCH,), jnp.float32),
                       pltpu.VMEM((_SCCH,), jnp.float32),
                       pltpu.SemaphoreType.DMA],
    )(pts_flat, xh, yh, zh, gidx)
    gp = gp_flat.reshape(B, S, K, D)
    gx3 = gx.reshape(B, S, K)
    gy3 = gy.reshape(B, S, K)
    gz3 = gz.reshape(B, S, K)

    xspec = pl.BlockSpec((1, S, K), lambda b, c: (b, 0, 0))
    out = pl.pallas_call(
        _fused_out_body,
        grid=(B, NCH),
        in_specs=[pl.BlockSpec((1, S, K, D), lambda b, c: (b, 0, 0, 0)),
                  xspec, xspec, xspec,
                  pl.BlockSpec((1, S, C), lambda b, c: (b, 0, 0)),
                  pl.BlockSpec((2, C), lambda b, c: (0, 0))],
        out_specs=pl.BlockSpec((1, QC, K, C + D), lambda b, c: (b, c, 0, 0)),
        out_shape=jax.ShapeDtypeStruct((B, S, K, C + D), jnp.float32),
        scratch_shapes=[pltpu.VMEM((S, C), jnp.float32),
                        pltpu.SMEM((1, 1), jnp.float32)],
    )(gp, gx3, gy3, gz3, nrows, ab)

    new_xyz = nrows[:, :, D:C]
    return new_xyz, out


def kernel(xyz, points, affine_alpha, affine_beta):
    B, N, _ = xyz.shape
    D = points.shape[2]
    S, K, QC = _S, _K, _QC
    C = D + 3

    xt = jnp.transpose(xyz, (0, 2, 1))
    comb = jnp.concatenate([points, xyz], axis=2)
    ab = jnp.concatenate([affine_alpha.reshape(1, C),
                          affine_beta.reshape(1, C)], axis=0)

    fidx = pl.pallas_call(
        _fps_body,
        out_shape=jax.ShapeDtypeStruct((B, 1, S), jnp.int32),
    )(xt)

    return _half_pipeline(xyz, points, comb, xt, fidx, ab, S, K, QC)


# speedup vs baseline: 1.2930x; 1.2032x over previous
import functools

import jax
import jax.numpy as jnp
from jax import lax
from jax.experimental import pallas as pl
from jax.experimental.pallas import tpu as pltpu
from jax.experimental.pallas import tpu_sc as plsc

_S = 512
_K = 32
_QC = 128
_SCCH = 128


def _fps_body(xt_ref, out_ref):
    B, _, N = xt_ref.shape
    S = out_ref.shape[2]
    x = xt_ref[:, 0, :]
    y = xt_ref[:, 1, :]
    z = xt_ref[:, 2, :]
    iota = jax.lax.broadcasted_iota(jnp.int32, (B, N), 1)
    sio = jax.lax.broadcasted_iota(jnp.int32, (B, S), 1)

    def body(i, carry):
        dists, nxt, acc = carry
        eq = iota == nxt
        px = jnp.sum(jnp.where(eq, x, 0.0), axis=1, keepdims=True)
        py = jnp.sum(jnp.where(eq, y, 0.0), axis=1, keepdims=True)
        pz = jnp.sum(jnp.where(eq, z, 0.0), axis=1, keepdims=True)
        dx = x - px
        dy = y - py
        dz = z - pz
        d = dx * dx + dy * dy + dz * dz
        dists = jnp.minimum(dists, d)
        nn = jnp.argmax(dists, axis=1, keepdims=True).astype(jnp.int32)
        acc = jnp.where(sio == i, nn, acc)
        return dists, nn, acc

    init = (jnp.full((B, N), jnp.inf, jnp.float32),
            jnp.zeros((B, 1), jnp.int32),
            jnp.zeros((B, S), jnp.int32))
    _, _, acc = jax.lax.fori_loop(1, S, body, init, unroll=4)
    out_ref[:, 0, :] = acc


def _knn_body(comb_ref, xt_ref, fidx_ref, nrows_ref, kidx_ref):
    _, N, C = comb_ref.shape
    S = nrows_ref.shape[1]
    K = kidx_ref.shape[2]
    comb = comb_ref[0]
    fidx = fidx_ref[0]
    niota = jax.lax.broadcasted_iota(jnp.int32, (N, S), 0)
    ohT = jnp.where(niota == fidx, 1.0, 0.0).astype(jnp.float32)
    nrows = jax.lax.dot_general(ohT, comb, (((0,), (0,)), ((), ())),
                                precision=jax.lax.Precision.HIGHEST,
                                preferred_element_type=jnp.float32)
    nrows_ref[0] = nrows
    x = xt_ref[0, 0:1, :]
    y = xt_ref[0, 1:2, :]
    z = xt_ref[0, 2:3, :]
    qx = nrows[:, C - 3:C - 2]
    qy = nrows[:, C - 2:C - 1]
    qz = nrows[:, C - 1:C]
    dxm = qx - x
    dym = qy - y
    dzm = qz - z
    d = dxm * dxm + dym * dym + dzm * dzm
    iota_n = jax.lax.broadcasted_iota(jnp.int32, (S, N), 1)
    kio = jax.lax.broadcasted_iota(jnp.int32, (S, K), 1)

    def body(j, carry):
        dcur, acc = carry
        e = jnp.argmin(dcur, axis=1, keepdims=True).astype(jnp.int32)
        acc = jnp.where(kio == j, e, acc)
        dcur = jnp.where(iota_n == e, jnp.inf, dcur)
        return dcur, acc

    _, acc = jax.lax.fori_loop(0, K, body, (d, jnp.zeros((S, K), jnp.int32)),
                               unroll=4)
    kidx_ref[0] = acc + pl.program_id(0) * N


def _sc_gather_body(pts_ref, xh_ref, yh_ref, zh_ref, gidx_ref,
                    gp_ref, gx_ref, gy_ref, gz_ref,
                    xt_v, yt_v, zt_v, idx_v, rows_v, xb_v, yb_v, zb_v, sem):
    R = gidx_ref.shape[0]
    ch = idx_v.shape[0]
    nc = 2
    nw = 32
    per_w = R // nw
    chunks = per_w // ch
    wid = lax.axis_index("s") * nc + lax.axis_index("c")

    pltpu.sync_copy(xh_ref, xt_v)
    pltpu.sync_copy(yh_ref, yt_v)
    pltpu.sync_copy(zh_ref, zt_v)

    def body(i, carry):
        base = wid * per_w + i * ch
        pltpu.sync_copy(gidx_ref.at[pl.ds(base, ch)], idx_v)
        cp = pltpu.async_copy(pts_ref.at[idx_v], rows_v, sem)
        for t in range(ch // 16):
            iv = idx_v[pl.ds(16 * t, 16)]
            xb_v[pl.ds(16 * t, 16)] = plsc.load_gather(xt_v, [iv])
            yb_v[pl.ds(16 * t, 16)] = plsc.load_gather(yt_v, [iv])
            zb_v[pl.ds(16 * t, 16)] = plsc.load_gather(zt_v, [iv])
        pltpu.sync_copy(xb_v, gx_ref.at[pl.ds(base, ch)])
        pltpu.sync_copy(yb_v, gy_ref.at[pl.ds(base, ch)])
        pltpu.sync_copy(zb_v, gz_ref.at[pl.ds(base, ch)])
        cp.wait()
        pltpu.sync_copy(rows_v, gp_ref.at[pl.ds(base, ch)])
        return carry

    jax.lax.fori_loop(0, chunks, body, 0)


def _stats_body(gp_ref, gx_ref, gy_ref, gz_ref, mean_ref, vpart_ref):
    _, QC, K, D = gp_ref.shape
    NCH = vpart_ref.shape[2]
    c = pl.program_id(1)
    gp = gp_ref[0]
    mup = jnp.mean(gp, axis=1)
    vp = gp - mup[:, None, :]
    s = jnp.sum(vp * vp)
    mus = []
    for r in (gx_ref, gy_ref, gz_ref):
        gc = r[0]
        muc = jnp.mean(gc, axis=1, keepdims=True)
        vc = gc - muc
        s = s + jnp.sum(vc * vc)
        mus.append(muc)
    mean_ref[0] = jnp.concatenate([mup] + mus, axis=1)
    lio = jax.lax.broadcasted_iota(jnp.int32, (1, NCH), 1)

    @pl.when(c == 0)
    def _():
        vpart_ref[0] = jnp.zeros((1, NCH), jnp.float32)

    vpart_ref[0] = vpart_ref[0] + jnp.where(lio == c, s, 0.0)


def _out_body(gp_ref, gx_ref, gy_ref, gz_ref, mean_ref, vparts_ref,
              nrows_ref, ab_ref, out_ref):
    _, QC, K, D = gp_ref.shape
    C = mean_ref.shape[2]
    NCH = vparts_ref.shape[2]
    M = NCH * QC * K * C
    s2 = jnp.sum(vparts_ref[0, 0, :])
    std = jnp.sqrt(s2 / (M - 1))
    scale = 1.0 / (std + 1e-5)
    mu = mean_ref[0]
    alpha = ab_ref[0:1, :]
    beta = ab_ref[1:2, :]
    normp = (gp_ref[0] - mu[:, None, 0:D]) * scale
    normp = normp * alpha[None, :, 0:D] + beta[None, :, 0:D]
    parts = [normp]
    for j, r in enumerate((gx_ref, gy_ref, gz_ref)):
        gc = r[0]
        nc = (gc - mu[:, D + j:D + j + 1]) * scale
        nc = nc * ab_ref[0:1, D + j:D + j + 1] + ab_ref[1:2, D + j:D + j + 1]
        parts.append(nc[:, :, None])
    rep = nrows_ref[0][:, 0:D]
    parts.append(jnp.broadcast_to(rep[:, None, :], (QC, K, D)))
    out_ref[0] = jnp.concatenate(parts, axis=-1)


def _fused_out_body(gp_ref, gx_ref, gy_ref, gz_ref, nrows_ref, ab_ref,
                    out_ref, mean_scr, scale_scr):
    _, S, K, D = gp_ref.shape
    C = nrows_ref.shape[2]
    QC = out_ref.shape[1]
    M = S * K * C
    c = pl.program_id(1)

    @pl.when(c == 0)
    def _():
        gp = gp_ref[0]
        mup = jnp.mean(gp, axis=1)
        vp = gp - mup[:, None, :]
        s2 = jnp.sum(vp * vp)
        mucs = []
        for r in (gx_ref, gy_ref, gz_ref):
            gc = r[0]
            muc = jnp.mean(gc, axis=1, keepdims=True)
            vc = gc - muc
            s2 = s2 + jnp.sum(vc * vc)
            mucs.append(muc)
        std = jnp.sqrt(s2 / (M - 1))
        mean_scr[:, :] = jnp.concatenate([mup] + mucs, axis=1)
        scale_scr[0, 0] = 1.0 / (std + 1e-5)

    scale = scale_scr[0, 0]
    ab = ab_ref[:, :]
    mu = mean_scr[pl.ds(QC * c, QC), :]
    gpc = gp_ref[0, pl.ds(QC * c, QC)]
    normp = (gpc - mu[:, None, 0:D]) * scale
    normp = normp * ab[0:1, None, 0:D] + ab[1:2, None, 0:D]
    parts = [normp]
    for j, r in enumerate((gx_ref, gy_ref, gz_ref)):
        gc = r[0, pl.ds(QC * c, QC)]
        nc = (gc - mu[:, D + j:D + j + 1]) * scale
        nc = nc * ab[0:1, D + j:D + j + 1] + ab[1:2, D + j:D + j + 1]
        parts.append(nc[:, :, None])
    rep = nrows_ref[0, pl.ds(QC * c, QC), 0:D]
    parts.append(jnp.broadcast_to(rep[:, None, :], (QC, K, D)))
    out_ref[0] = jnp.concatenate(parts, axis=-1)


def _half_pipeline(xyz, points, comb, xt, fidx, ab, S, K, QC):
    B, N, _ = xyz.shape
    D = points.shape[2]
    C = D + 3
    NCH = S // QC

    nrows, kidx = pl.pallas_call(
        _knn_body,
        grid=(B,),
        in_specs=[pl.BlockSpec((1, N, C), lambda b: (b, 0, 0)),
                  pl.BlockSpec((1, 3, N), lambda b: (b, 0, 0)),
                  pl.BlockSpec((1, 1, S), lambda b: (b, 0, 0))],
        out_specs=[pl.BlockSpec((1, S, C), lambda b: (b, 0, 0)),
                   pl.BlockSpec((1, S, K), lambda b: (b, 0, 0))],
        out_shape=[jax.ShapeDtypeStruct((B, S, C), jnp.float32),
                   jax.ShapeDtypeStruct((B, S, K), jnp.int32)],
    )(comb, xt, fidx)

    R = B * S * K
    pts_flat = points.reshape(B * N, D)
    xh = xyz[:, :, 0].reshape(B * N)
    yh = xyz[:, :, 1].reshape(B * N)
    zh = xyz[:, :, 2].reshape(B * N)
    gidx = kidx.reshape(R)
    mesh = plsc.VectorSubcoreMesh(core_axis_name="c", subcore_axis_name="s")
    gp_flat, gx, gy, gz = pl.kernel(
        _sc_gather_body,
        mesh=mesh,
        compiler_params=pltpu.CompilerParams(needs_layout_passes=False),
        out_type=[jax.ShapeDtypeStruct((R, D), jnp.float32),
                  jax.ShapeDtypeStruct((R,), jnp.float32),
                  jax.ShapeDtypeStruct((R,), jnp.float32),
                  jax.ShapeDtypeStruct((R,), jnp.float32)],
        scratch_types=[pltpu.VMEM((B * N,), jnp.float32),
                       pltpu.VMEM((B * N,), jnp.float32),
                       pltpu.VMEM((B * N,), jnp.float32),
                       pltpu.VMEM((_SCCH,), jnp.int32),
                       pltpu.VMEM((_SCCH, D), jnp.float32),
                       pltpu.VMEM((_SCCH,), jnp.float32),
                       pltpu.VMEM((_SCCH,), jnp.float32),
                       pltpu.VMEM((_SCCH,), jnp.float32),
                       pltpu.SemaphoreType.DMA],
    )(pts_flat, xh, yh, zh, gidx)
    gp = gp_flat.reshape(B, S, K, D)
    gx3 = gx.reshape(B, S, K)
    gy3 = gy.reshape(B, S, K)
    gz3 = gz.reshape(B, S, K)

    xspec = pl.BlockSpec((1, S, K), lambda b, c: (b, 0, 0))
    out = pl.pallas_call(
        _fused_out_body,
        grid=(B, NCH),
        in_specs=[pl.BlockSpec((1, S, K, D), lambda b, c: (b, 0, 0, 0)),
                  xspec, xspec, xspec,
                  pl.BlockSpec((1, S, C), lambda b, c: (b, 0, 0)),
                  pl.BlockSpec((2, C), lambda b, c: (0, 0))],
        out_specs=pl.BlockSpec((1, QC, K, C + D), lambda b, c: (b, c, 0, 0)),
        out_shape=jax.ShapeDtypeStruct((B, S, K, C + D), jnp.float32),
        scratch_shapes=[pltpu.VMEM((S, C), jnp.float32),
                        pltpu.SMEM((1, 1), jnp.float32)],
    )(gp, gx3, gy3, gz3, nrows, ab)

    new_xyz = nrows[:, :, D:C]
    return new_xyz, out


def kernel(xyz, points, affine_alpha, affine_beta):
    B, N, _ = xyz.shape
    D = points.shape[2]
    S, K, QC = _S, _K, _QC
    C = D + 3

    xt = jnp.transpose(xyz, (0, 2, 1))
    comb = jnp.concatenate([points, xyz], axis=2)
    ab = jnp.concatenate([affine_alpha.reshape(1, C),
                          affine_beta.reshape(1, C)], axis=0)

    fidx = pl.pallas_call(
        _fps_body,
        out_shape=jax.ShapeDtypeStruct((B, 1, S), jnp.int32),
    )(xt)

    return _half_pipeline(xyz, points, comb, xt, fidx, ab, S, K, QC)


# speedup vs baseline: 1.3693x; 1.0590x over previous
import functools

import jax
import jax.numpy as jnp
from jax import lax
from jax.experimental import pallas as pl
from jax.experimental.pallas import tpu as pltpu
from jax.experimental.pallas import tpu_sc as plsc

_S = 512
_K = 32
_QC = 128
_SCCH = 128


def _fps_body(xt_ref, out_ref):
    B, _, N = xt_ref.shape
    S = out_ref.shape[2]
    x = xt_ref[:, 0, :]
    y = xt_ref[:, 1, :]
    z = xt_ref[:, 2, :]
    iota = jax.lax.broadcasted_iota(jnp.int32, (B, N), 1)
    sio = jax.lax.broadcasted_iota(jnp.int32, (B, S), 1)

    def body(i, carry):
        dists, nxt, acc = carry
        eq = iota == nxt
        px = jnp.sum(jnp.where(eq, x, 0.0), axis=1, keepdims=True)
        py = jnp.sum(jnp.where(eq, y, 0.0), axis=1, keepdims=True)
        pz = jnp.sum(jnp.where(eq, z, 0.0), axis=1, keepdims=True)
        dx = x - px
        dy = y - py
        dz = z - pz
        d = dx * dx + dy * dy + dz * dz
        dists = jnp.minimum(dists, d)
        nn = jnp.argmax(dists, axis=1, keepdims=True).astype(jnp.int32)
        acc = jnp.where(sio == i, nn, acc)
        return dists, nn, acc

    init = (jnp.full((B, N), jnp.inf, jnp.float32),
            jnp.zeros((B, 1), jnp.int32),
            jnp.zeros((B, S), jnp.int32))
    _, _, acc = jax.lax.fori_loop(1, S, body, init, unroll=8)
    out_ref[:, 0, :] = acc


def _knn_body(comb_ref, xt_ref, fidx_ref, nrows_ref, kidx_ref):
    _, N, C = comb_ref.shape
    S = nrows_ref.shape[1]
    K = kidx_ref.shape[2]
    comb = comb_ref[0]
    fidx = fidx_ref[0]
    niota = jax.lax.broadcasted_iota(jnp.int32, (N, S), 0)
    ohT = jnp.where(niota == fidx, 1.0, 0.0).astype(jnp.float32)
    nrows = jax.lax.dot_general(ohT, comb, (((0,), (0,)), ((), ())),
                                precision=jax.lax.Precision.HIGHEST,
                                preferred_element_type=jnp.float32)
    nrows_ref[0] = nrows
    x = xt_ref[0, 0:1, :]
    y = xt_ref[0, 1:2, :]
    z = xt_ref[0, 2:3, :]
    qx = nrows[:, C - 3:C - 2]
    qy = nrows[:, C - 2:C - 1]
    qz = nrows[:, C - 1:C]
    dxm = qx - x
    dym = qy - y
    dzm = qz - z
    d = dxm * dxm + dym * dym + dzm * dzm
    iota_n = jax.lax.broadcasted_iota(jnp.int32, (S, N), 1)
    kio = jax.lax.broadcasted_iota(jnp.int32, (S, K), 1)

    def body(j, carry):
        dcur, acc = carry
        e = jnp.argmin(dcur, axis=1, keepdims=True).astype(jnp.int32)
        acc = jnp.where(kio == j, e, acc)
        dcur = jnp.where(iota_n == e, jnp.inf, dcur)
        return dcur, acc

    _, acc = jax.lax.fori_loop(0, K, body, (d, jnp.zeros((S, K), jnp.int32)),
                               unroll=8)
    kidx_ref[0] = acc + pl.program_id(0) * N


def _sc_gather_body(pts_ref, xh_ref, yh_ref, zh_ref, gidx_ref,
                    gp_ref, gx_ref, gy_ref, gz_ref,
                    xt_v, yt_v, zt_v, idx_v, rows_v, xb_v, yb_v, zb_v, sem):
    R = gidx_ref.shape[0]
    ch = idx_v.shape[0]
    nc = 2
    nw = 32
    per_w = R // nw
    chunks = per_w // ch
    wid = lax.axis_index("s") * nc + lax.axis_index("c")

    pltpu.sync_copy(xh_ref, xt_v)
    pltpu.sync_copy(yh_ref, yt_v)
    pltpu.sync_copy(zh_ref, zt_v)

    def body(i, carry):
        base = wid * per_w + i * ch
        pltpu.sync_copy(gidx_ref.at[pl.ds(base, ch)], idx_v)
        cp = pltpu.async_copy(pts_ref.at[idx_v], rows_v, sem)
        for t in range(ch // 16):
            iv = idx_v[pl.ds(16 * t, 16)]
            xb_v[pl.ds(16 * t, 16)] = plsc.load_gather(xt_v, [iv])
            yb_v[pl.ds(16 * t, 16)] = plsc.load_gather(yt_v, [iv])
            zb_v[pl.ds(16 * t, 16)] = plsc.load_gather(zt_v, [iv])
        pltpu.sync_copy(xb_v, gx_ref.at[pl.ds(base, ch)])
        pltpu.sync_copy(yb_v, gy_ref.at[pl.ds(base, ch)])
        pltpu.sync_copy(zb_v, gz_ref.at[pl.ds(base, ch)])
        cp.wait()
        pltpu.sync_copy(rows_v, gp_ref.at[pl.ds(base, ch)])
        return carry

    jax.lax.fori_loop(0, chunks, body, 0)


def _stats_body(gp_ref, gx_ref, gy_ref, gz_ref, mean_ref, vpart_ref):
    _, QC, K, D = gp_ref.shape
    NCH = vpart_ref.shape[2]
    c = pl.program_id(1)
    gp = gp_ref[0]
    mup = jnp.mean(gp, axis=1)
    vp = gp - mup[:, None, :]
    s = jnp.sum(vp * vp)
    mus = []
    for r in (gx_ref, gy_ref, gz_ref):
        gc = r[0]
        muc = jnp.mean(gc, axis=1, keepdims=True)
        vc = gc - muc
        s = s + jnp.sum(vc * vc)
        mus.append(muc)
    mean_ref[0] = jnp.concatenate([mup] + mus, axis=1)
    lio = jax.lax.broadcasted_iota(jnp.int32, (1, NCH), 1)

    @pl.when(c == 0)
    def _():
        vpart_ref[0] = jnp.zeros((1, NCH), jnp.float32)

    vpart_ref[0] = vpart_ref[0] + jnp.where(lio == c, s, 0.0)


def _out_body(gp_ref, gx_ref, gy_ref, gz_ref, mean_ref, vparts_ref,
              nrows_ref, ab_ref, out_ref):
    _, QC, K, D = gp_ref.shape
    C = mean_ref.shape[2]
    NCH = vparts_ref.shape[2]
    M = NCH * QC * K * C
    s2 = jnp.sum(vparts_ref[0, 0, :])
    std = jnp.sqrt(s2 / (M - 1))
    scale = 1.0 / (std + 1e-5)
    mu = mean_ref[0]
    alpha = ab_ref[0:1, :]
    beta = ab_ref[1:2, :]
    normp = (gp_ref[0] - mu[:, None, 0:D]) * scale
    normp = normp * alpha[None, :, 0:D] + beta[None, :, 0:D]
    parts = [normp]
    for j, r in enumerate((gx_ref, gy_ref, gz_ref)):
        gc = r[0]
        nc = (gc - mu[:, D + j:D + j + 1]) * scale
        nc = nc * ab_ref[0:1, D + j:D + j + 1] + ab_ref[1:2, D + j:D + j + 1]
        parts.append(nc[:, :, None])
    rep = nrows_ref[0][:, 0:D]
    parts.append(jnp.broadcast_to(rep[:, None, :], (QC, K, D)))
    out_ref[0] = jnp.concatenate(parts, axis=-1)


def _fused_out_body(gp_ref, gx_ref, gy_ref, gz_ref, nrows_ref, ab_ref,
                    out_ref, mean_scr, scale_scr):
    _, S, K, D = gp_ref.shape
    C = nrows_ref.shape[2]
    QC = out_ref.shape[1]
    M = S * K * C
    c = pl.program_id(1)

    @pl.when(c == 0)
    def _():
        gp = gp_ref[0]
        mup = jnp.mean(gp, axis=1)
        vp = gp - mup[:, None, :]
        s2 = jnp.sum(vp * vp)
        mucs = []
        for r in (gx_ref, gy_ref, gz_ref):
            gc = r[0]
            muc = jnp.mean(gc, axis=1, keepdims=True)
            vc = gc - muc
            s2 = s2 + jnp.sum(vc * vc)
            mucs.append(muc)
        std = jnp.sqrt(s2 / (M - 1))
        mean_scr[:, :] = jnp.concatenate([mup] + mucs, axis=1)
        scale_scr[0, 0] = 1.0 / (std + 1e-5)

    scale = scale_scr[0, 0]
    ab = ab_ref[:, :]
    mu = mean_scr[pl.ds(QC * c, QC), :]
    gpc = gp_ref[0, pl.ds(QC * c, QC)]
    normp = (gpc - mu[:, None, 0:D]) * scale
    normp = normp * ab[0:1, None, 0:D] + ab[1:2, None, 0:D]
    parts = [normp]
    for j, r in enumerate((gx_ref, gy_ref, gz_ref)):
        gc = r[0, pl.ds(QC * c, QC)]
        nc = (gc - mu[:, D + j:D + j + 1]) * scale
        nc = nc * ab[0:1, D + j:D + j + 1] + ab[1:2, D + j:D + j + 1]
        parts.append(nc[:, :, None])
    rep = nrows_ref[0, pl.ds(QC * c, QC), 0:D]
    parts.append(jnp.broadcast_to(rep[:, None, :], (QC, K, D)))
    out_ref[0] = jnp.concatenate(parts, axis=-1)


def _half_pipeline(xyz, points, comb, xt, fidx, ab, S, K, QC):
    B, N, _ = xyz.shape
    D = points.shape[2]
    C = D + 3
    NCH = S // QC

    nrows, kidx = pl.pallas_call(
        _knn_body,
        grid=(B,),
        in_specs=[pl.BlockSpec((1, N, C), lambda b: (b, 0, 0)),
                  pl.BlockSpec((1, 3, N), lambda b: (b, 0, 0)),
                  pl.BlockSpec((1, 1, S), lambda b: (b, 0, 0))],
        out_specs=[pl.BlockSpec((1, S, C), lambda b: (b, 0, 0)),
                   pl.BlockSpec((1, S, K), lambda b: (b, 0, 0))],
        out_shape=[jax.ShapeDtypeStruct((B, S, C), jnp.float32),
                   jax.ShapeDtypeStruct((B, S, K), jnp.int32)],
    )(comb, xt, fidx)

    R = B * S * K
    pts_flat = points.reshape(B * N, D)
    xh = xyz[:, :, 0].reshape(B * N)
    yh = xyz[:, :, 1].reshape(B * N)
    zh = xyz[:, :, 2].reshape(B * N)
    gidx = kidx.reshape(R)
    mesh = plsc.VectorSubcoreMesh(core_axis_name="c", subcore_axis_name="s")
    gp_flat, gx, gy, gz = pl.kernel(
        _sc_gather_body,
        mesh=mesh,
        compiler_params=pltpu.CompilerParams(needs_layout_passes=False),
        out_type=[jax.ShapeDtypeStruct((R, D), jnp.float32),
                  jax.ShapeDtypeStruct((R,), jnp.float32),
                  jax.ShapeDtypeStruct((R,), jnp.float32),
                  jax.ShapeDtypeStruct((R,), jnp.float32)],
        scratch_types=[pltpu.VMEM((B * N,), jnp.float32),
                       pltpu.VMEM((B * N,), jnp.float32),
                       pltpu.VMEM((B * N,), jnp.float32),
                       pltpu.VMEM((_SCCH,), jnp.int32),
                       pltpu.VMEM((_SCCH, D), jnp.float32),
                       pltpu.VMEM((_SCCH,), jnp.float32),
                       pltpu.VMEM((_SCCH,), jnp.float32),
                       pltpu.VMEM((_SCCH,), jnp.float32),
                       pltpu.SemaphoreType.DMA],
    )(pts_flat, xh, yh, zh, gidx)
    gp = gp_flat.reshape(B, S, K, D)
    gx3 = gx.reshape(B, S, K)
    gy3 = gy.reshape(B, S, K)
    gz3 = gz.reshape(B, S, K)

    xspec = pl.BlockSpec((1, S, K), lambda b, c: (b, 0, 0))
    out = pl.pallas_call(
        _fused_out_body,
        grid=(B, NCH),
        in_specs=[pl.BlockSpec((1, S, K, D), lambda b, c: (b, 0, 0, 0)),
                  xspec, xspec, xspec,
                  pl.BlockSpec((1, S, C), lambda b, c: (b, 0, 0)),
                  pl.BlockSpec((2, C), lambda b, c: (0, 0))],
        out_specs=pl.BlockSpec((1, QC, K, C + D), lambda b, c: (b, c, 0, 0)),
        out_shape=jax.ShapeDtypeStruct((B, S, K, C + D), jnp.float32),
        scratch_shapes=[pltpu.VMEM((S, C), jnp.float32),
                        pltpu.SMEM((1, 1), jnp.float32)],
    )(gp, gx3, gy3, gz3, nrows, ab)

    new_xyz = nrows[:, :, D:C]
    return new_xyz, out


def kernel(xyz, points, affine_alpha, affine_beta):
    B, N, _ = xyz.shape
    D = points.shape[2]
    S, K, QC = _S, _K, _QC
    C = D + 3

    xt = jnp.transpose(xyz, (0, 2, 1))
    comb = jnp.concatenate([points, xyz], axis=2)
    ab = jnp.concatenate([affine_alpha.reshape(1, C),
                          affine_beta.reshape(1, C)], axis=0)

    fidx = pl.pallas_call(
        _fps_body,
        out_shape=jax.ShapeDtypeStruct((B, 1, S), jnp.int32),
    )(xt)

    return _half_pipeline(xyz, points, comb, xt, fidx, ab, S, K, QC)


# speedup vs baseline: 1.4018x; 1.0238x over previous
import functools

import jax
import jax.numpy as jnp
from jax import lax
from jax.experimental import pallas as pl
from jax.experimental.pallas import tpu as pltpu
from jax.experimental.pallas import tpu_sc as plsc

_S = 512
_K = 32
_QC = 128
_SCCH = 128


def _fps_body(xt_ref, out_ref):
    B, _, N = xt_ref.shape
    S = out_ref.shape[2]
    x = xt_ref[:, 0, :]
    y = xt_ref[:, 1, :]
    z = xt_ref[:, 2, :]
    iota = jax.lax.broadcasted_iota(jnp.int32, (B, N), 1)
    sio = jax.lax.broadcasted_iota(jnp.int32, (B, S), 1)

    def body(i, carry):
        dists, nxt, acc = carry
        eq = iota == nxt
        px = jnp.sum(jnp.where(eq, x, 0.0), axis=1, keepdims=True)
        py = jnp.sum(jnp.where(eq, y, 0.0), axis=1, keepdims=True)
        pz = jnp.sum(jnp.where(eq, z, 0.0), axis=1, keepdims=True)
        dx = x - px
        dy = y - py
        dz = z - pz
        d = dx * dx + dy * dy + dz * dz
        dists = jnp.minimum(dists, d)
        nn = jnp.argmax(dists, axis=1, keepdims=True).astype(jnp.int32)
        acc = jnp.where(sio == i, nn, acc)
        return dists, nn, acc

    init = (jnp.full((B, N), jnp.inf, jnp.float32),
            jnp.zeros((B, 1), jnp.int32),
            jnp.zeros((B, S), jnp.int32))
    _, _, acc = jax.lax.fori_loop(1, S, body, init, unroll=16)
    out_ref[:, 0, :] = acc


def _knn_body(comb_ref, xt_ref, fidx_ref, nrows_ref, kidx_ref):
    _, N, C = comb_ref.shape
    S = nrows_ref.shape[1]
    K = kidx_ref.shape[2]
    comb = comb_ref[0]
    fidx = fidx_ref[0]
    niota = jax.lax.broadcasted_iota(jnp.int32, (N, S), 0)
    ohT = jnp.where(niota == fidx, 1.0, 0.0).astype(jnp.float32)
    nrows = jax.lax.dot_general(ohT, comb, (((0,), (0,)), ((), ())),
                                precision=jax.lax.Precision.HIGHEST,
                                preferred_element_type=jnp.float32)
    nrows_ref[0] = nrows
    x = xt_ref[0, 0:1, :]
    y = xt_ref[0, 1:2, :]
    z = xt_ref[0, 2:3, :]
    qx = nrows[:, C - 3:C - 2]
    qy = nrows[:, C - 2:C - 1]
    qz = nrows[:, C - 1:C]
    dxm = qx - x
    dym = qy - y
    dzm = qz - z
    d = dxm * dxm + dym * dym + dzm * dzm
    iota_n = jax.lax.broadcasted_iota(jnp.int32, (S, N), 1)
    kio = jax.lax.broadcasted_iota(jnp.int32, (S, K), 1)

    def body(j, carry):
        dcur, acc = carry
        e = jnp.argmin(dcur, axis=1, keepdims=True).astype(jnp.int32)
        acc = jnp.where(kio == j, e, acc)
        dcur = jnp.where(iota_n == e, jnp.inf, dcur)
        return dcur, acc

    _, acc = jax.lax.fori_loop(0, K, body, (d, jnp.zeros((S, K), jnp.int32)),
                               unroll=16)
    kidx_ref[0] = acc + pl.program_id(0) * N


def _sc_gather_body(pts_ref, xh_ref, yh_ref, zh_ref, gidx_ref,
                    gp_ref, gx_ref, gy_ref, gz_ref,
                    xt_v, yt_v, zt_v, idx_v, rows_v, xb_v, yb_v, zb_v, sem):
    R = gidx_ref.shape[0]
    ch = idx_v.shape[0]
    nc = 2
    nw = 32
    per_w = R // nw
    chunks = per_w // ch
    wid = lax.axis_index("s") * nc + lax.axis_index("c")

    pltpu.sync_copy(xh_ref, xt_v)
    pltpu.sync_copy(yh_ref, yt_v)
    pltpu.sync_copy(zh_ref, zt_v)

    def body(i, carry):
        base = wid * per_w + i * ch
        pltpu.sync_copy(gidx_ref.at[pl.ds(base, ch)], idx_v)
        cp = pltpu.async_copy(pts_ref.at[idx_v], rows_v, sem)
        for t in range(ch // 16):
            iv = idx_v[pl.ds(16 * t, 16)]
            xb_v[pl.ds(16 * t, 16)] = plsc.load_gather(xt_v, [iv])
            yb_v[pl.ds(16 * t, 16)] = plsc.load_gather(yt_v, [iv])
            zb_v[pl.ds(16 * t, 16)] = plsc.load_gather(zt_v, [iv])
        pltpu.sync_copy(xb_v, gx_ref.at[pl.ds(base, ch)])
        pltpu.sync_copy(yb_v, gy_ref.at[pl.ds(base, ch)])
        pltpu.sync_copy(zb_v, gz_ref.at[pl.ds(base, ch)])
        cp.wait()
        pltpu.sync_copy(rows_v, gp_ref.at[pl.ds(base, ch)])
        return carry

    jax.lax.fori_loop(0, chunks, body, 0)


def _stats_body(gp_ref, gx_ref, gy_ref, gz_ref, mean_ref, vpart_ref):
    _, QC, K, D = gp_ref.shape
    NCH = vpart_ref.shape[2]
    c = pl.program_id(1)
    gp = gp_ref[0]
    mup = jnp.mean(gp, axis=1)
    vp = gp - mup[:, None, :]
    s = jnp.sum(vp * vp)
    mus = []
    for r in (gx_ref, gy_ref, gz_ref):
        gc = r[0]
        muc = jnp.mean(gc, axis=1, keepdims=True)
        vc = gc - muc
        s = s + jnp.sum(vc * vc)
        mus.append(muc)
    mean_ref[0] = jnp.concatenate([mup] + mus, axis=1)
    lio = jax.lax.broadcasted_iota(jnp.int32, (1, NCH), 1)

    @pl.when(c == 0)
    def _():
        vpart_ref[0] = jnp.zeros((1, NCH), jnp.float32)

    vpart_ref[0] = vpart_ref[0] + jnp.where(lio == c, s, 0.0)


def _out_body(gp_ref, gx_ref, gy_ref, gz_ref, mean_ref, vparts_ref,
              nrows_ref, ab_ref, out_ref):
    _, QC, K, D = gp_ref.shape
    C = mean_ref.shape[2]
    NCH = vparts_ref.shape[2]
    M = NCH * QC * K * C
    s2 = jnp.sum(vparts_ref[0, 0, :])
    std = jnp.sqrt(s2 / (M - 1))
    scale = 1.0 / (std + 1e-5)
    mu = mean_ref[0]
    alpha = ab_ref[0:1, :]
    beta = ab_ref[1:2, :]
    normp = (gp_ref[0] - mu[:, None, 0:D]) * scale
    normp = normp * alpha[None, :, 0:D] + beta[None, :, 0:D]
    parts = [normp]
    for j, r in enumerate((gx_ref, gy_ref, gz_ref)):
        gc = r[0]
        nc = (gc - mu[:, D + j:D + j + 1]) * scale
        nc = nc * ab_ref[0:1, D + j:D + j + 1] + ab_ref[1:2, D + j:D + j + 1]
        parts.append(nc[:, :, None])
    rep = nrows_ref[0][:, 0:D]
    parts.append(jnp.broadcast_to(rep[:, None, :], (QC, K, D)))
    out_ref[0] = jnp.concatenate(parts, axis=-1)


def _fused_out_body(gp_ref, gx_ref, gy_ref, gz_ref, nrows_ref, ab_ref,
                    out_ref, mean_scr, scale_scr):
    _, S, K, D = gp_ref.shape
    C = nrows_ref.shape[2]
    QC = out_ref.shape[1]
    M = S * K * C
    c = pl.program_id(1)

    @pl.when(c == 0)
    def _():
        gp = gp_ref[0]
        mup = jnp.mean(gp, axis=1)
        vp = gp - mup[:, None, :]
        s2 = jnp.sum(vp * vp)
        mucs = []
        for r in (gx_ref, gy_ref, gz_ref):
            gc = r[0]
            muc = jnp.mean(gc, axis=1, keepdims=True)
            vc = gc - muc
            s2 = s2 + jnp.sum(vc * vc)
            mucs.append(muc)
        std = jnp.sqrt(s2 / (M - 1))
        mean_scr[:, :] = jnp.concatenate([mup] + mucs, axis=1)
        scale_scr[0, 0] = 1.0 / (std + 1e-5)

    scale = scale_scr[0, 0]
    ab = ab_ref[:, :]
    mu = mean_scr[pl.ds(QC * c, QC), :]
    gpc = gp_ref[0, pl.ds(QC * c, QC)]
    normp = (gpc - mu[:, None, 0:D]) * scale
    normp = normp * ab[0:1, None, 0:D] + ab[1:2, None, 0:D]
    parts = [normp]
    for j, r in enumerate((gx_ref, gy_ref, gz_ref)):
        gc = r[0, pl.ds(QC * c, QC)]
        nc = (gc - mu[:, D + j:D + j + 1]) * scale
        nc = nc * ab[0:1, D + j:D + j + 1] + ab[1:2, D + j:D + j + 1]
        parts.append(nc[:, :, None])
    rep = nrows_ref[0, pl.ds(QC * c, QC), 0:D]
    parts.append(jnp.broadcast_to(rep[:, None, :], (QC, K, D)))
    out_ref[0] = jnp.concatenate(parts, axis=-1)


def _half_pipeline(xyz, points, comb, xt, fidx, ab, S, K, QC):
    B, N, _ = xyz.shape
    D = points.shape[2]
    C = D + 3
    NCH = S // QC

    nrows, kidx = pl.pallas_call(
        _knn_body,
        grid=(B,),
        in_specs=[pl.BlockSpec((1, N, C), lambda b: (b, 0, 0)),
                  pl.BlockSpec((1, 3, N), lambda b: (b, 0, 0)),
                  pl.BlockSpec((1, 1, S), lambda b: (b, 0, 0))],
        out_specs=[pl.BlockSpec((1, S, C), lambda b: (b, 0, 0)),
                   pl.BlockSpec((1, S, K), lambda b: (b, 0, 0))],
        out_shape=[jax.ShapeDtypeStruct((B, S, C), jnp.float32),
                   jax.ShapeDtypeStruct((B, S, K), jnp.int32)],
    )(comb, xt, fidx)

    R = B * S * K
    pts_flat = points.reshape(B * N, D)
    xh = xyz[:, :, 0].reshape(B * N)
    yh = xyz[:, :, 1].reshape(B * N)
    zh = xyz[:, :, 2].reshape(B * N)
    gidx = kidx.reshape(R)
    mesh = plsc.VectorSubcoreMesh(core_axis_name="c", subcore_axis_name="s")
    gp_flat, gx, gy, gz = pl.kernel(
        _sc_gather_body,
        mesh=mesh,
        compiler_params=pltpu.CompilerParams(needs_layout_passes=False),
        out_type=[jax.ShapeDtypeStruct((R, D), jnp.float32),
                  jax.ShapeDtypeStruct((R,), jnp.float32),
                  jax.ShapeDtypeStruct((R,), jnp.float32),
                  jax.ShapeDtypeStruct((R,), jnp.float32)],
        scratch_types=[pltpu.VMEM((B * N,), jnp.float32),
                       pltpu.VMEM((B * N,), jnp.float32),
                       pltpu.VMEM((B * N,), jnp.float32),
                       pltpu.VMEM((_SCCH,), jnp.int32),
                       pltpu.VMEM((_SCCH, D), jnp.float32),
                       pltpu.VMEM((_SCCH,), jnp.float32),
                       pltpu.VMEM((_SCCH,), jnp.float32),
                       pltpu.VMEM((_SCCH,), jnp.float32),
                       pltpu.SemaphoreType.DMA],
    )(pts_flat, xh, yh, zh, gidx)
    gp = gp_flat.reshape(B, S, K, D)
    gx3 = gx.reshape(B, S, K)
    gy3 = gy.reshape(B, S, K)
    gz3 = gz.reshape(B, S, K)

    xspec = pl.BlockSpec((1, S, K), lambda b, c: (b, 0, 0))
    out = pl.pallas_call(
        _fused_out_body,
        grid=(B, NCH),
        in_specs=[pl.BlockSpec((1, S, K, D), lambda b, c: (b, 0, 0, 0)),
                  xspec, xspec, xspec,
                  pl.BlockSpec((1, S, C), lambda b, c: (b, 0, 0)),
                  pl.BlockSpec((2, C), lambda b, c: (0, 0))],
        out_specs=pl.BlockSpec((1, QC, K, C + D), lambda b, c: (b, c, 0, 0)),
        out_shape=jax.ShapeDtypeStruct((B, S, K, C + D), jnp.float32),
        scratch_shapes=[pltpu.VMEM((S, C), jnp.float32),
                        pltpu.SMEM((1, 1), jnp.float32)],
    )(gp, gx3, gy3, gz3, nrows, ab)

    new_xyz = nrows[:, :, D:C]
    return new_xyz, out


def kernel(xyz, points, affine_alpha, affine_beta):
    B, N, _ = xyz.shape
    D = points.shape[2]
    S, K, QC = _S, _K, _QC
    C = D + 3

    xt = jnp.transpose(xyz, (0, 2, 1))
    comb = jnp.concatenate([points, xyz], axis=2)
    ab = jnp.concatenate([affine_alpha.reshape(1, C),
                          affine_beta.reshape(1, C)], axis=0)

    fidx = pl.pallas_call(
        _fps_body,
        out_shape=jax.ShapeDtypeStruct((B, 1, S), jnp.int32),
    )(xt)

    return _half_pipeline(xyz, points, comb, xt, fidx, ab, S, K, QC)
